# baseline (device time: 39619 ns/iter reference)
import jax
import jax.numpy as jnp
from jax import lax
from jax.experimental import pallas as pl
from jax.experimental.pallas import tpu as pltpu

B, SQ, H, D = 8, 1, 8, 64
SKV = 512
SCALE = D ** -0.5
F32 = jnp.float32


def _body(q_ref, k_ref, v_ref, out_ref,
          loc_o, loc_m, loc_l, peer_o, peer_m, peer_l,
          send_sems, recv_sems):
    b = pl.program_id(0)

    q = q_ref[0, 0, :, :]
    k = k_ref[0]
    v = v_ref[0]
    s = jnp.sum(q[None, :, :] * k, axis=-1) * SCALE
    m = jnp.max(s, axis=0)
    p = jnp.exp(s - m[None, :])
    l = jnp.sum(p, axis=0)
    o = jnp.sum(p[:, :, None] * v, axis=0)
    loc_o[b] = o
    loc_m[b, :] = m
    loc_l[b, :] = l

    @pl.when(b == B - 1)
    def _():
        my_x = lax.axis_index("x")
        my_y = lax.axis_index("y")
        nbr = (my_x, 1 - my_y)

        barrier = pltpu.get_barrier_semaphore()
        pl.semaphore_signal(barrier, inc=1, device_id=nbr,
                            device_id_type=pl.DeviceIdType.MESH)
        pl.semaphore_wait(barrier, 1)

        copies = [
            pltpu.make_async_remote_copy(
                src_ref=src, dst_ref=dst,
                send_sem=send_sems.at[i], recv_sem=recv_sems.at[i],
                device_id=nbr, device_id_type=pl.DeviceIdType.MESH,
            )
            for i, (src, dst) in enumerate(
                [(loc_o, peer_o), (loc_m, peer_m), (loc_l, peer_l)]
            )
        ]
        for c in copies:
            c.start()
        for c in copies:
            c.wait()

        m_new = jnp.maximum(loc_m[...], peer_m[...])
        a_loc = jnp.exp(loc_m[...] - m_new)
        a_peer = jnp.exp(peer_m[...] - m_new)
        l_new = a_loc * loc_l[...] + a_peer * peer_l[...]
        o_new = (a_loc[:, :, None] * loc_o[...]
                 + a_peer[:, :, None] * peer_o[...]) / l_new[:, :, None]
        out_ref[...] = o_new[:, None, :, :]


def kernel(Q, K, V):
    return pl.pallas_call(
        _body,
        grid=(B,),
        out_shape=jax.ShapeDtypeStruct((B, SQ, H, D), F32),
        in_specs=[
            pl.BlockSpec((1, SQ, H, D), lambda b: (b, 0, 0, 0)),
            pl.BlockSpec((1, SKV, H, D), lambda b: (b, 0, 0, 0)),
            pl.BlockSpec((1, SKV, H, D), lambda b: (b, 0, 0, 0)),
        ],
        out_specs=pl.BlockSpec((B, SQ, H, D), lambda b: (0, 0, 0, 0)),
        scratch_shapes=[
            pltpu.VMEM((B, H, D), F32),
            pltpu.VMEM((B, H), F32),
            pltpu.VMEM((B, H), F32),
            pltpu.VMEM((B, H, D), F32),
            pltpu.VMEM((B, H), F32),
            pltpu.VMEM((B, H), F32),
            pltpu.SemaphoreType.DMA((3,)),
            pltpu.SemaphoreType.DMA((3,)),
        ],
        compiler_params=pltpu.CompilerParams(collective_id=0),
    )(Q, K, V)
